# Initial kernel scaffold; baseline (speedup 1.0000x reference)
import jax
import jax.numpy as jnp
from jax.experimental import pallas as pl


def kernel(pts, verts, face_normals, face_centers, uv, vf, uv_f, mesh_f):
    raise NotImplementedError("write your pallas kernel here")


# speedup vs baseline: 2.4000x; 2.4000x over previous
import functools

import jax
import jax.numpy as jnp
from jax import lax
from jax.experimental import pallas as pl
from jax.experimental.pallas import tpu as pltpu
from jax.experimental.pallas import tpu_sc as plsc

N_VERTS = 6890
N_FACES = 13776
N_UV = 7576

Q = 8192
QB = 512
VB = 1024
VPAD = 7168

NC = 2
NS = 16
NW = NC * NS
QW = Q // NW
NCH = QW // 16

_FQ = 0
_FFC = 3
_FFN = 21
_FABC = 24
_FUV = 33
_FO = 39
_FSD = 42
_FTOT = 43

_IVI = 0
_IVF = 1
_IFI = 7
_IMF = 8
_IUVF = 11
_IFI2 = 14
_ITOT = 15


def _nn_body(qx_ref, qy_ref, qz_ref, vm_ref, out_ref, mn_ref, mi_ref):
    vb = pl.program_id(1)
    vx = vm_ref[0:1, :]
    vy = vm_ref[1:2, :]
    vz = vm_ref[2:3, :]
    qx = qx_ref[...]
    qy = qy_ref[...]
    qz = qz_ref[...]
    bfr = lambda t: t.astype(jnp.bfloat16).astype(jnp.float32)
    q2x, q2y, q2z = bfr(2.0 * qx), bfr(2.0 * qy), bfr(2.0 * qz)
    vbx, vby, vbz = bfr(vx), bfr(vy), bfr(vz)
    dot = q2x * vbx + q2y * vby + q2z * vbz
    qq = qx * qx + qy * qy + qz * qz
    vv = vx * vx + vy * vy + vz * vz
    d2 = (qq + vv) - dot
    m = jnp.min(d2, axis=1, keepdims=True)
    cols = lax.broadcasted_iota(jnp.int32, (QB, VB), 1) + vb * VB
    idx = jnp.min(jnp.where(d2 == m, cols, jnp.int32(2 ** 30)),
                  axis=1, keepdims=True)

    @pl.when(vb == 0)
    def _():
        mn_ref[...] = m
        mi_ref[...] = idx

    @pl.when(vb > 0)
    def _():
        upd = m < mn_ref[...]
        mn_ref[...] = jnp.where(upd, m, mn_ref[...])
        mi_ref[...] = jnp.where(upd, idx, mi_ref[...])

    @pl.when(vb == pl.num_programs(1) - 1)
    def _():
        out_ref[...] = mi_ref[...]


def _nearest_vertex(qx, qy, qz, vmatT):
    grid = (Q // QB, VPAD // VB)
    return pl.pallas_call(
        _nn_body,
        grid=grid,
        in_specs=[
            pl.BlockSpec((QB, 1), lambda qb, vb: (qb, 0)),
            pl.BlockSpec((QB, 1), lambda qb, vb: (qb, 0)),
            pl.BlockSpec((QB, 1), lambda qb, vb: (qb, 0)),
            pl.BlockSpec((8, VB), lambda qb, vb: (0, vb)),
        ],
        out_specs=pl.BlockSpec((QB, 1), lambda qb, vb: (qb, 0)),
        out_shape=jax.ShapeDtypeStruct((Q, 1), jnp.int32),
        scratch_shapes=[
            pltpu.VMEM((QB, 1), jnp.float32),
            pltpu.VMEM((QB, 1), jnp.int32),
        ],
        compiler_params=pltpu.CompilerParams(
            dimension_semantics=("parallel", "arbitrary"),
        ),
    )(qx, qy, qz, vmatT)


def _sqrt16(x):
    xb = lax.bitcast_convert_type(x, jnp.int32)
    r = lax.bitcast_convert_type(jnp.int32(0x5F3759DF) - (xb >> 1),
                                 jnp.float32)
    for _ in range(3):
        r = r * (1.5 - 0.5 * x * r * r)
    return jnp.where(x <= 0.0, 0.0, x * r)


def _safe16(d):
    return jnp.where(d == 0.0, 1.0, d)


def _gather_128(table_hbm, ibuf, islot, fbuf_or_ibuf, dslot, sem):
    hs = []
    for c in range(QW // 128):
        idx = ibuf.at[pl.ds(islot * QW + c * 128, 128)]
        dst = fbuf_or_ibuf.at[pl.ds(dslot * QW + c * 128, 128)]
        hs.append(pltpu.async_copy(table_hbm.at[idx], dst, sem))
    return hs


def _sc_body(qx_hbm, qy_hbm, qz_hbm, vi_hbm,
             vf0, vf1, vf2, vf3, vf4, vf5,
             fcx, fcy, fcz, fnx, fny, fnz,
             mf0, mf1, mf2, uvf0, uvf1, uvf2,
             vx_t, vy_t, vz_t, uvu_t, uvv_t,
             out_u_hbm, out_v_hbm, out_d_hbm, out_sd_hbm, fi_stage_hbm,
             fbuf, ibuf, sem):
    wid = lax.axis_index("c") * NS + lax.axis_index("s")
    base = wid * QW

    pltpu.sync_copy(vi_hbm.at[pl.ds(base, QW)], ibuf.at[pl.ds(_IVI * QW, QW)])
    pltpu.sync_copy(qx_hbm.at[pl.ds(base, QW)], fbuf.at[pl.ds(_FQ * QW, QW)])
    pltpu.sync_copy(qy_hbm.at[pl.ds(base, QW)],
                    fbuf.at[pl.ds((_FQ + 1) * QW, QW)])
    pltpu.sync_copy(qz_hbm.at[pl.ds(base, QW)],
                    fbuf.at[pl.ds((_FQ + 2) * QW, QW)])

    hs = []
    for k, tab in enumerate((vf0, vf1, vf2, vf3, vf4, vf5)):
        hs += _gather_128(tab, ibuf, _IVI, ibuf, _IVF + k, sem)
    for h in hs:
        h.wait()

    hs = []
    for k in range(6):
        for c, tab in enumerate((fcx, fcy, fcz)):
            hs += _gather_128(tab, ibuf, _IVF + k, fbuf, _FFC + k * 3 + c,
                              sem)
    for h in hs:
        h.wait()

    def pick_face(j, _):
        o = j * 16
        qx = fbuf[pl.ds(_FQ * QW + o, 16)]
        qy = fbuf[pl.ds((_FQ + 1) * QW + o, 16)]
        qz = fbuf[pl.ds((_FQ + 2) * QW + o, 16)]
        bestd = None
        bestk = None
        for k in range(6):
            s = _FFC + k * 3
            dx = qx - fbuf[pl.ds(s * QW + o, 16)]
            dy = qy - fbuf[pl.ds((s + 1) * QW + o, 16)]
            dz = qz - fbuf[pl.ds((s + 2) * QW + o, 16)]
            dk = dx * dx + dy * dy + dz * dz
            if k == 0:
                bestd, bestk = dk, jnp.zeros((16,), jnp.int32)
            else:
                m = dk < bestd
                bestd = jnp.where(m, dk, bestd)
                bestk = jnp.where(m, k, bestk)
        face = ibuf[pl.ds((_IVF + 0) * QW + o, 16)]
        for k in range(1, 6):
            face = jnp.where(bestk == k, ibuf[pl.ds((_IVF + k) * QW + o, 16)],
                             face)
        ibuf[pl.ds(_IFI * QW + o, 16)] = face
        return _
    lax.fori_loop(0, NCH, pick_face, None)

    pltpu.sync_copy(ibuf.at[pl.ds(_IFI * QW, QW)],
                    fi_stage_hbm.at[pl.ds(base, QW)])
    pltpu.sync_copy(fi_stage_hbm.at[pl.ds(base, QW)],
                    ibuf.at[pl.ds(_IFI2 * QW, QW)])

    hs = []
    for s, tab in enumerate((mf0, mf1, mf2)):
        hs += _gather_128(tab, ibuf, _IFI2, ibuf, _IMF + s, sem)
    for s, tab in enumerate((uvf0, uvf1, uvf2)):
        hs += _gather_128(tab, ibuf, _IFI2, ibuf, _IUVF + s, sem)
    for c, tab in enumerate((fnx, fny, fnz)):
        hs += _gather_128(tab, ibuf, _IFI2, fbuf, _FFN + c, sem)
    for h in hs:
        h.wait()

    hs = []
    for s in range(3):
        for c, tab in enumerate((vx_t, vy_t, vz_t)):
            hs += _gather_128(tab, ibuf, _IMF + s, fbuf, _FABC + s * 3 + c,
                              sem)
        for c, tab in enumerate((uvu_t, uvv_t)):
            hs += _gather_128(tab, ibuf, _IUVF + s, fbuf, _FUV + s * 2 + c,
                              sem)
    for h in hs:
        h.wait()

    def main_math(j, _):
        o = j * 16
        f = lambda slot: fbuf[pl.ds(slot * QW + o, 16)]
        qx, qy, qz = f(_FQ), f(_FQ + 1), f(_FQ + 2)
        ax, ay, az = f(_FABC), f(_FABC + 1), f(_FABC + 2)
        bx, by, bz = f(_FABC + 3), f(_FABC + 4), f(_FABC + 5)
        cx, cy, cz = f(_FABC + 6), f(_FABC + 7), f(_FABC + 8)

        abx, aby, abz = bx - ax, by - ay, bz - az
        acx, acy, acz = cx - ax, cy - ay, cz - az
        apx, apy, apz = qx - ax, qy - ay, qz - az
        d1 = abx * apx + aby * apy + abz * apz
        d2 = acx * apx + acy * apy + acz * apz
        bpx, bpy, bpz = qx - bx, qy - by, qz - bz
        d3 = abx * bpx + aby * bpy + abz * bpz
        d4 = acx * bpx + acy * bpy + acz * bpz
        cpx, cpy, cpz = qx - cx, qy - cy, qz - cz
        d5 = abx * cpx + aby * cpy + abz * cpz
        d6 = acx * cpx + acy * cpy + acz * cpz
        vc = d1 * d4 - d3 * d2
        vb_ = d5 * d2 - d1 * d6
        va = d3 * d6 - d5 * d4
        c1 = (d1 <= 0.0) & (d2 <= 0.0)
        c2 = (d3 >= 0.0) & (d4 <= d3)
        c3 = (d6 >= 0.0) & (d5 < d6)
        c4 = (vc <= 0.0) & (d1 >= 0.0) & (d3 <= 0.0)
        c5 = (vb_ <= 0.0) & (d2 >= 0.0) & (d6 <= 0.0)
        c6 = (va <= 0.0) & (d4 >= d3) & (d6 <= d5)
        cany = c1 | c2 | c3 | c4 | c5 | c6
        v1 = d1 / _safe16(d1 - d3)
        w1 = d2 / _safe16(d2 - d6)
        w2 = (d4 - d3) / _safe16((d4 - d3) + (d5 - d6))
        inv = 1.0 / _safe16(va + vb_ + vc)
        v = vb_ * inv
        w = vc * inv

        px = jnp.zeros((16,), jnp.float32)
        py = jnp.zeros((16,), jnp.float32)
        pz = jnp.zeros((16,), jnp.float32)
        px = jnp.where(c1, ax, px); py = jnp.where(c1, ay, py); pz = jnp.where(c1, az, pz)
        px = jnp.where(c2, bx, px); py = jnp.where(c2, by, py); pz = jnp.where(c2, bz, pz)
        px = jnp.where(c3, cx, px); py = jnp.where(c3, cy, py); pz = jnp.where(c3, cz, pz)
        px = jnp.where(c4, ax + v1 * abx, px); py = jnp.where(c4, ay + v1 * aby, py); pz = jnp.where(c4, az + v1 * abz, pz)
        px = jnp.where(c5, ax + w1 * acx, px); py = jnp.where(c5, ay + w1 * acy, py); pz = jnp.where(c5, az + w1 * acz, pz)
        bcx, bcy, bcz = cx - bx, cy - by, cz - bz
        px = jnp.where(c6, bx + w2 * bcx, px); py = jnp.where(c6, by + w2 * bcy, py); pz = jnp.where(c6, bz + w2 * bcz, pz)
        px = jnp.where(cany, px, ax + abx * v + w * acx)
        py = jnp.where(cany, py, ay + aby * v + w * acy)
        pz = jnp.where(cany, pz, az + abz * v + w * acz)

        z16 = jnp.zeros((16,), jnp.float32)
        b0, b1, b2 = z16, z16, z16
        b0 = jnp.where(c1, 1.0, b0)
        b1 = jnp.where(c2, 1.0, b1)
        b2 = jnp.where(c3, 1.0, b2)
        b0 = jnp.where(c4, 1.0 - v1, b0); b1 = jnp.where(c4, v1, b1)
        b0 = jnp.where(c5, 1.0 - w1, b0); b2 = jnp.where(c5, w1, b2)
        b1 = jnp.where(c6, 1.0 - w2, b1); b2 = jnp.where(c6, w2, b2)
        b0 = jnp.where(cany, b0, 1.0 - v - w)
        b1 = jnp.where(cany, b1, v)
        b2 = jnp.where(cany, b2, w)

        ex, ey, ez = px - qx, py - qy, pz - qz
        distance = _sqrt16(ex * ex + ey * ey + ez * ez)

        nx_, ny_, nz_ = f(_FFN), f(_FFN + 1), f(_FFN + 2)
        dot = nx_ * (qx - px) + ny_ * (qy - py) + nz_ * (qz - pz)
        sd = jnp.sign(dot) * distance

        u0, v0 = f(_FUV), f(_FUV + 1)
        u1_, v1_ = f(_FUV + 2), f(_FUV + 3)
        u2_, v2_ = f(_FUV + 4), f(_FUV + 5)
        ru = b0 * u0 + b1 * u1_ + b2 * u2_
        rv = b0 * v0 + b1 * v1_ + b2 * v2_

        dist = 1.0 / (1.0 + jnp.exp(-10.0 * sd))

        one = jnp.float32(1.0)
        zero = jnp.float32(0.0)
        fbuf[pl.ds(_FO * QW + o, 16)] = jnp.clip(ru, zero, one)
        fbuf[pl.ds((_FO + 1) * QW + o, 16)] = jnp.clip(rv, zero, one)
        fbuf[pl.ds((_FO + 2) * QW + o, 16)] = jnp.clip(dist, zero, one)
        fbuf[pl.ds(_FSD * QW + o, 16)] = sd
        return _
    lax.fori_loop(0, NCH, main_math, None)

    for c, dst in enumerate((out_u_hbm, out_v_hbm, out_d_hbm)):
        pltpu.sync_copy(fbuf.at[pl.ds((_FO + c) * QW, QW)],
                        dst.at[pl.ds(base, QW)])
    pltpu.sync_copy(fbuf.at[pl.ds(_FSD * QW, QW)],
                    out_sd_hbm.at[pl.ds(base, QW)])


def _sc_stage(args):
    mesh = plsc.VectorSubcoreMesh(core_axis_name="c", subcore_axis_name="s",
                                  num_cores=NC, num_subcores=NS)
    run = functools.partial(
        pl.kernel,
        out_type=(jax.ShapeDtypeStruct((Q,), jnp.float32),
                  jax.ShapeDtypeStruct((Q,), jnp.float32),
                  jax.ShapeDtypeStruct((Q,), jnp.float32),
                  jax.ShapeDtypeStruct((Q,), jnp.float32),
                  jax.ShapeDtypeStruct((Q,), jnp.int32)),
        mesh=mesh,
        scratch_types=[
            pltpu.VMEM((_FTOT * QW,), jnp.float32),
            pltpu.VMEM((_ITOT * QW,), jnp.int32),
            pltpu.SemaphoreType.DMA,
        ],
    )(_sc_body)
    return run(*args)


def kernel(pts, verts, face_normals, face_centers, uv, vf, uv_f, mesh_f):
    q2 = pts.reshape(-1, 3)
    v = verts[0]

    qxc = q2[:, 0:1]
    qyc = q2[:, 1:2]
    qzc = q2[:, 2:3]
    vpad = jnp.pad(v, ((0, VPAD - N_VERTS), (0, 0)), constant_values=1e6)
    vmatT = jnp.pad(vpad.T, ((0, 5), (0, 0)))

    vi = _nearest_vertex(qxc, qyc, qzc, vmatT)[:, 0]

    fc = face_centers[0]
    fn = face_normals[0]
    args = (
        qxc[:, 0], qyc[:, 0], qzc[:, 0], vi,
        vf[:, 0], vf[:, 1], vf[:, 2], vf[:, 3], vf[:, 4], vf[:, 5],
        fc[:, 0], fc[:, 1], fc[:, 2], fn[:, 0], fn[:, 1], fn[:, 2],
        mesh_f[:, 0], mesh_f[:, 1], mesh_f[:, 2],
        uv_f[:, 0], uv_f[:, 1], uv_f[:, 2],
        v[:, 0], v[:, 1], v[:, 2], uv[:, 0], uv[:, 1],
    )
    out_u, out_v, out_d, out_sd, _ = _sc_stage(args)
    return jnp.stack([out_u, out_v, out_d], axis=-1), out_sd
